# baseline (speedup 1.0000x reference)
import functools

import jax
import jax.numpy as jnp
from jax import lax
from jax.experimental import pallas as pl
from jax.experimental.pallas import tpu as pltpu
from jax.experimental.pallas import tpu_sc as plsc

D = 768
LANES = 16
VREGS_PER_ROW = D // LANES

NUM_CORES = 2
NUM_SUBCORES = 16
NW = NUM_CORES * NUM_SUBCORES

W = 32
WINS_PER_CHUNK = 28


def _emb_body(ids_hbm, pid_hbm, tok_tab, pos_tab, out_hbm,
              idx_v, pidx_v, tok_v, pos_tile, sidx_v, sem_t, sem_o,
              *, bsz, seq, toks_per_w):
    wid = lax.axis_index("s") * NUM_CORES + lax.axis_index("c")
    tok0 = wid * toks_per_w
    chunk_toks = WINS_PER_CHUNK * W
    n_chunks = toks_per_w // chunk_toks

    pltpu.sync_copy(
        pos_tab.at[pl.ds(pl.multiple_of(wid * seq * D, LANES), seq * D)],
        pos_tile)

    def gathers(i, buf):
        return (
            pltpu.make_async_copy(
                tok_tab.at[idx_v.at[pl.ds(i * W, W)]], tok_v.at[buf], sem_t),
        )

    def scatter(buf):
        return pltpu.make_async_copy(
            tok_v.at[buf], out_hbm.at[sidx_v.at[buf]], sem_o)

    def chunk_body(c, carry):
        chunk_tok = tok0 + c * chunk_toks
        pltpu.sync_copy(ids_hbm.at[pl.ds(chunk_tok, chunk_toks)], idx_v)
        pltpu.sync_copy(pid_hbm.at[pl.ds(chunk_tok, chunk_toks)], pidx_v)
        for g in gathers(0, 0):
            g.start()

        def body(i, carry2):
            buf = lax.rem(i, 2)
            nbuf = lax.rem(i + 1, 2)

            @pl.when(i >= 1)
            def _():
                scatter(nbuf).wait()

            @pl.when(i + 1 < WINS_PER_CHUNK)
            def _():
                for g in gathers(i + 1, nbuf):
                    g.start()

            (gt,) = gathers(i, buf)
            gt.wait()

            for k in range(W // LANES):
                pvec = pidx_v[pl.ds(i * W + k * LANES, LANES)]
                for r2 in range(LANES):
                    r = k * LANES + r2
                    p0 = pl.multiple_of(pvec[r2] * D, LANES)
                    for cc in range(VREGS_PER_ROW):
                        sl = pl.ds(cc * LANES, LANES)
                        tok_v[buf, r, sl] = tok_v[buf, r, sl] + pos_tile[
                            pl.ds(p0 + cc * LANES, LANES)]

            win_tok = chunk_tok + i * W
            for k in range(W // LANES):
                t = lax.broadcasted_iota(jnp.int32, (LANES,), 0) + (
                    win_tok + k * LANES)
                s = lax.rem(t, seq)
                b = lax.div(t, seq)
                sidx_v[buf, pl.ds(k * LANES, LANES)] = s * bsz + b

            scatter(buf).start()
            return carry2

        lax.fori_loop(0, WINS_PER_CHUNK, body, 0)
        scatter(lax.rem(WINS_PER_CHUNK - 1, 2)).wait()
        return carry

    lax.fori_loop(0, n_chunks, chunk_body, 0)


def kernel(input_ids, position_ids, token_table, position_table):
    bsz, seq = input_ids.shape
    n = bsz * seq
    toks_per_w = n // NW
    assert toks_per_w * NW == n
    assert toks_per_w % (WINS_PER_CHUNK * W) == 0

    ids = input_ids.astype(jnp.int32).reshape(n)
    pos_rep = jnp.tile(position_table, (NW, 1)).reshape(-1)
    pid = position_ids.astype(jnp.int32).reshape(n)

    mesh = plsc.VectorSubcoreMesh(core_axis_name="c", subcore_axis_name="s")
    run = pl.kernel(
        functools.partial(_emb_body, bsz=bsz, seq=seq,
                          toks_per_w=toks_per_w),
        mesh=mesh,
        out_type=jax.ShapeDtypeStruct((n, D), jnp.float32),
        scratch_types=[
            pltpu.VMEM((WINS_PER_CHUNK * W,), jnp.int32),
            pltpu.VMEM((WINS_PER_CHUNK * W,), jnp.int32),
            pltpu.VMEM((2, W, D), jnp.float32),
            pltpu.VMEM((seq * D,), jnp.float32),
            pltpu.VMEM((2, W), jnp.int32),
            pltpu.SemaphoreType.DMA,
            pltpu.SemaphoreType.DMA,
        ],
    )
    out_sm = run(ids, pid, token_table, pos_rep)
    return out_sm.reshape(seq, bsz, D).transpose(1, 0, 2)
